# Initial kernel scaffold; baseline (speedup 1.0000x reference)
import jax
import jax.numpy as jnp
from jax.experimental import pallas as pl


def kernel(edge_index, x):
    raise NotImplementedError("write your pallas kernel here")


# speedup vs baseline: 32.4716x; 32.4716x over previous
import functools

import jax
import jax.numpy as jnp
from jax import lax
from jax.experimental import pallas as pl
from jax.experimental.pallas import tpu as pltpu
from jax.experimental.pallas import tpu_sc as plsc

N_NODES = 10000
N_EDGES = 160000
D_FEAT = 256

NW = 32
NBINS = 10016
EDGES_PAD = 160256
EPT = EDGES_PAD // NW
PAD = EDGES_PAD - N_EDGES

_mesh = plsc.VectorSubcoreMesh(core_axis_name="c", subcore_axis_name="s")


@functools.partial(
    pl.kernel,
    mesh=_mesh,
    out_type=jax.ShapeDtypeStruct((NW, NBINS), jnp.int32),
    scratch_types=[
        pltpu.VMEM((EPT,), jnp.int32),
        pltpu.VMEM((NBINS,), jnp.int32),
    ],
    compiler_params=pltpu.CompilerParams(needs_layout_passes=False),
)
def _degree_kernel(col_hbm, out_hbm, idx_v, counts_v):
    wid = lax.axis_index("s") * 2 + lax.axis_index("c")
    pltpu.sync_copy(col_hbm.at[pl.ds(wid * EPT, EPT)], idx_v)

    def zero_body(i, carry):
        counts_v[pl.ds(i * 16, 16)] = jnp.zeros((16,), jnp.int32)
        return carry

    lax.fori_loop(0, NBINS // 16, zero_body, 0)

    ones = jnp.ones((16,), jnp.int32)

    def hist_body(i, carry):
        idx = idx_v[pl.ds(i * 16, 16)]
        plsc.addupdate_scatter(counts_v, [idx], ones)
        return carry

    lax.fori_loop(0, EPT // 16, hist_body, 0)

    pltpu.sync_copy(counts_v, out_hbm.at[wid])


_ROWS = 2048


def _scale_body(cnt_ref, x_ref, out_ref):
    deg = jnp.sum(cnt_ref[...], axis=0).astype(jnp.float32)
    out_ref[...] = x_ref[...] * deg[:, None]


def _scale(counts, x):
    return pl.pallas_call(
        _scale_body,
        grid=(pl.cdiv(N_NODES, _ROWS),),
        in_specs=[
            pl.BlockSpec((NW, _ROWS), lambda i: (0, i)),
            pl.BlockSpec((_ROWS, D_FEAT), lambda i: (i, 0)),
        ],
        out_specs=pl.BlockSpec((_ROWS, D_FEAT), lambda i: (i, 0)),
        out_shape=jax.ShapeDtypeStruct((N_NODES, D_FEAT), jnp.float32),
    )(counts, x)


@jax.jit
def kernel(edge_index, x):
    col = edge_index[1]
    col_p = jnp.concatenate([col, jnp.full((PAD,), N_NODES, jnp.int32)])
    counts = _degree_kernel(col_p)
    return _scale(counts, x)
